# baseline (speedup 1.0000x reference)
import jax
import jax.numpy as jnp
from jax.experimental import pallas as pl
from jax.experimental.pallas import tpu as pltpu

_CHUNK = 512
_SL = 256
_NORM_PERSISTENT = 1024.0


def _qkproj_kernel(q_ref, k_ref, m_acc, mb_acc, out_ref, n_acc):
    i = pl.program_id(0)

    @pl.when(i == 0)
    def _init():
        n_acc[0, 0] = _NORM_PERSISTENT

    dim = q_ref.shape[1]
    n_sl = dim // _SL

    kbs = []
    ss = None
    out = None
    scores = None
    for s in range(n_sl):
        sl = pl.ds(s * _SL, _SL)
        ks = k_ref[:, sl]
        qb = q_ref[:, sl].astype(jnp.bfloat16)
        kb = ks.astype(jnp.bfloat16)
        kbs.append(kb)
        part_ss = jnp.sum(ks * ks, axis=1, keepdims=True)
        ss = part_ss if ss is None else ss + part_ss
        d = jax.lax.dot_general(qb, mb_acc[:, sl], (((1,), (1,)), ((), ())),
                                preferred_element_type=jnp.float32)
        out = d if out is None else out + d
        d = jax.lax.dot_general(qb, kb, (((1,), (1,)), ((), ())),
                                preferred_element_type=jnp.float32)
        scores = d if scores is None else scores + d

    row = jax.lax.broadcasted_iota(jnp.int32, (_CHUNK, _CHUNK), 0)
    col = jax.lax.broadcasted_iota(jnp.int32, (_CHUNK, _CHUNK), 1)
    causal = (col <= row)
    scores = jnp.where(causal, scores, 0.0).astype(jnp.bfloat16)

    csum = jnp.dot(causal.astype(jnp.float32), ss,
                   preferred_element_type=jnp.float32)
    norms = jnp.maximum(n_acc[0, 0] + csum, 1e-8)
    n_acc[0, 0] = n_acc[0, 0] + jnp.sum(ss)

    for s in range(n_sl):
        sl = pl.ds(s * _SL, _SL)
        d = jax.lax.dot_general(scores, kbs[s], (((1,), (0,)), ((), ())),
                                preferred_element_type=jnp.float32)
        out_ref[:, sl] = (out[:, s * _SL:(s + 1) * _SL] + d) / norms

    for a in range(n_sl):
        sla = pl.ds(a * _SL, _SL)
        for b in range(n_sl):
            slb = pl.ds(b * _SL, _SL)
            d = jax.lax.dot_general(kbs[a], kbs[b], (((0,), (0,)), ((), ())),
                                    preferred_element_type=jnp.float32)
            blk = m_acc[sla, slb] + d
            m_acc[sla, slb] = blk
            mb_acc[sla, slb] = blk.astype(jnp.bfloat16)


def kernel(queries, keys, m_persistent):
    t_len, dim = queries.shape
    n_chunks = t_len // _CHUNK
    m_persistent_bf16 = m_persistent.astype(jnp.bfloat16)
    return pl.pallas_call(
        _qkproj_kernel,
        out_shape=jax.ShapeDtypeStruct((t_len, dim), jnp.float32),
        grid=(n_chunks,),
        in_specs=[
            pl.BlockSpec((_CHUNK, dim), lambda i: (i, 0)),
            pl.BlockSpec((_CHUNK, dim), lambda i: (i, 0)),
            pl.BlockSpec((dim, dim), lambda i: (0, 0)),
            pl.BlockSpec((dim, dim), lambda i: (0, 0)),
        ],
        out_specs=pl.BlockSpec((_CHUNK, dim), lambda i: (i, 0)),
        scratch_shapes=[
            pltpu.SMEM((1, 1), jnp.float32),
        ],
        compiler_params=pltpu.CompilerParams(
            dimension_semantics=("arbitrary",),
        ),
        name="qkprojection",
    )(queries, keys, m_persistent, m_persistent_bf16)


# speedup vs baseline: 1.0739x; 1.0739x over previous
import jax
import jax.numpy as jnp
from jax.experimental import pallas as pl
from jax.experimental.pallas import tpu as pltpu

_CHUNK = 512
_SL = 256
_NORM_PERSISTENT = 1024.0


def _qkproj_kernel(q_ref, k_ref, m_acc, out_ref, mb_acc, n_acc):
    i = pl.program_id(0)

    @pl.when(i == 0)
    def _init():
        mb_acc[...] = m_acc[...].astype(jnp.bfloat16)
        n_acc[0, 0] = _NORM_PERSISTENT

    dim = q_ref.shape[1]
    n_sl = dim // _SL

    kbs = []
    ss = None
    out = None
    scores = None
    for s in range(n_sl):
        sl = pl.ds(s * _SL, _SL)
        ks = k_ref[:, sl]
        qb = q_ref[:, sl].astype(jnp.bfloat16)
        kb = ks.astype(jnp.bfloat16)
        kbs.append(kb)
        part_ss = jnp.sum(ks * ks, axis=1, keepdims=True)
        ss = part_ss if ss is None else ss + part_ss
        d = jax.lax.dot_general(qb, mb_acc[:, sl], (((1,), (1,)), ((), ())),
                                preferred_element_type=jnp.float32)
        out = d if out is None else out + d
        d = jax.lax.dot_general(qb, kb, (((1,), (1,)), ((), ())),
                                preferred_element_type=jnp.float32)
        scores = d if scores is None else scores + d

    row = jax.lax.broadcasted_iota(jnp.int32, (_CHUNK, _CHUNK), 0)
    col = jax.lax.broadcasted_iota(jnp.int32, (_CHUNK, _CHUNK), 1)
    causal = (col <= row)
    scores = jnp.where(causal, scores, 0.0).astype(jnp.bfloat16)

    csum = jnp.dot(causal.astype(jnp.float32), ss,
                   preferred_element_type=jnp.float32)
    norms = jnp.maximum(n_acc[0, 0] + csum, 1e-8)
    n_acc[0, 0] = n_acc[0, 0] + jnp.sum(ss)

    for s in range(n_sl):
        sl = pl.ds(s * _SL, _SL)
        d = jax.lax.dot_general(scores, kbs[s], (((1,), (0,)), ((), ())),
                                preferred_element_type=jnp.float32)
        out_ref[:, sl] = (out[:, s * _SL:(s + 1) * _SL] + d) / norms

    for a in range(n_sl):
        sla = pl.ds(a * _SL, _SL)
        for b in range(n_sl):
            slb = pl.ds(b * _SL, _SL)
            d = jax.lax.dot_general(kbs[a], kbs[b], (((0,), (0,)), ((), ())),
                                    preferred_element_type=jnp.float32)
            blk = m_acc[sla, slb] + d
            m_acc[sla, slb] = blk
            mb_acc[sla, slb] = blk.astype(jnp.bfloat16)


def kernel(queries, keys, m_persistent):
    t_len, dim = queries.shape
    n_chunks = t_len // _CHUNK
    return pl.pallas_call(
        _qkproj_kernel,
        out_shape=jax.ShapeDtypeStruct((t_len, dim), jnp.float32),
        grid=(n_chunks,),
        in_specs=[
            pl.BlockSpec((_CHUNK, dim), lambda i: (i, 0)),
            pl.BlockSpec((_CHUNK, dim), lambda i: (i, 0)),
            pl.BlockSpec((dim, dim), lambda i: (0, 0)),
        ],
        out_specs=pl.BlockSpec((_CHUNK, dim), lambda i: (i, 0)),
        scratch_shapes=[
            pltpu.VMEM((dim, dim), jnp.bfloat16),
            pltpu.SMEM((1, 1), jnp.float32),
        ],
        compiler_params=pltpu.CompilerParams(
            dimension_semantics=("arbitrary",),
        ),
        name="qkprojection",
    )(queries, keys, m_persistent)


# speedup vs baseline: 1.0811x; 1.0067x over previous
import jax
import jax.numpy as jnp
from jax.experimental import pallas as pl
from jax.experimental.pallas import tpu as pltpu

_CHUNK = 512
_SL = 256
_NORM_PERSISTENT = 1024.0


def _qkproj_kernel(q_ref, k_ref, m_acc, out_ref, n_acc):
    i = pl.program_id(0)

    @pl.when(i == 0)
    def _init():
        n_acc[0, 0] = _NORM_PERSISTENT

    dim = q_ref.shape[1]
    n_sl = dim // _SL

    ks_parts = []
    ss = None
    out = None
    scores = None
    for s in range(n_sl):
        sl = pl.ds(s * _SL, _SL)
        qs = q_ref[:, sl]
        ks = k_ref[:, sl]
        ks_parts.append(ks)
        part_ss = jnp.sum(ks * ks, axis=1, keepdims=True)
        ss = part_ss if ss is None else ss + part_ss
        d = jax.lax.dot_general(qs, m_acc[:, sl], (((1,), (1,)), ((), ())),
                                preferred_element_type=jnp.float32)
        out = d if out is None else out + d
        d = jax.lax.dot_general(qs, ks, (((1,), (1,)), ((), ())),
                                preferred_element_type=jnp.float32)
        scores = d if scores is None else scores + d

    row = jax.lax.broadcasted_iota(jnp.int32, (_CHUNK, _CHUNK), 0)
    col = jax.lax.broadcasted_iota(jnp.int32, (_CHUNK, _CHUNK), 1)
    causal = (col <= row)
    scores = jnp.where(causal, scores, 0.0)

    csum = jnp.dot(causal.astype(jnp.float32), ss,
                   preferred_element_type=jnp.float32)
    norms = jnp.maximum(n_acc[0, 0] + csum, 1e-8)
    n_acc[0, 0] = n_acc[0, 0] + jnp.sum(ss)

    for s in range(n_sl):
        sl = pl.ds(s * _SL, _SL)
        d = jax.lax.dot_general(scores, ks_parts[s], (((1,), (0,)), ((), ())),
                                preferred_element_type=jnp.float32)
        out_ref[:, sl] = (out[:, s * _SL:(s + 1) * _SL] + d) / norms

    for a in range(n_sl):
        sla = pl.ds(a * _SL, _SL)
        for b in range(n_sl):
            slb = pl.ds(b * _SL, _SL)
            d = jax.lax.dot_general(ks_parts[a], ks_parts[b],
                                    (((0,), (0,)), ((), ())),
                                    preferred_element_type=jnp.float32)
            m_acc[sla, slb] = m_acc[sla, slb] + d


def kernel(queries, keys, m_persistent):
    t_len, dim = queries.shape
    n_chunks = t_len // _CHUNK
    return pl.pallas_call(
        _qkproj_kernel,
        out_shape=jax.ShapeDtypeStruct((t_len, dim), jnp.float32),
        grid=(n_chunks,),
        in_specs=[
            pl.BlockSpec((_CHUNK, dim), lambda i: (i, 0)),
            pl.BlockSpec((_CHUNK, dim), lambda i: (i, 0)),
            pl.BlockSpec((dim, dim), lambda i: (0, 0)),
        ],
        out_specs=pl.BlockSpec((_CHUNK, dim), lambda i: (i, 0)),
        scratch_shapes=[
            pltpu.SMEM((1, 1), jnp.float32),
        ],
        compiler_params=pltpu.CompilerParams(
            dimension_semantics=("arbitrary",),
        ),
        name="qkprojection",
    )(queries, keys, m_persistent)
